# baseline (speedup 1.0000x reference)
import functools

import jax
import jax.numpy as jnp
from jax import lax
from jax.experimental import pallas as pl
from jax.experimental.pallas import tpu as pltpu
from jax.experimental.pallas import tpu_sc as plsc

NC = 2
NS = 16
NW = NC * NS
LANES = 16
CH = 128
NBUF = 2


def _dotT(a, b):
    return lax.dot_general(a, b, (((1,), (1,)), ((), ())),
                           preferred_element_type=jnp.float32)


def _tc_pre_body(x_ref, wl_ref, wr_ref, bl_ref, m_ref, r_ref):
    xb = x_ref[...]
    m_ref[...] = _dotT(xb, wl_ref[...])
    r_ref[...] = _dotT(xb, wr_ref[...]) + bl_ref[...]


def _tc_mid_body(p_ref, cnt_ref, r_ref, wl_ref, wr_ref, bl_ref, m_ref, rn_ref):
    cb = cnt_ref[...]
    cnt = cb[0, :, 0:1] + cb[1, :, 0:1]
    inv = 1.0 / jnp.maximum(cnt, 1.0)
    h = (p_ref[0] + p_ref[1]) * inv + r_ref[...]
    h = jnp.maximum(h, 0.0)
    m_ref[...] = _dotT(h, wl_ref[...])
    rn_ref[...] = _dotT(h, wr_ref[...]) + bl_ref[...]


def _tc_fin_body(p_ref, cnt_ref, r_ref, o_ref):
    cb = cnt_ref[...]
    cnt = cb[0, :, 0:1] + cb[1, :, 0:1]
    inv = 1.0 / jnp.maximum(cnt, 1.0)
    o_ref[...] = (p_ref[0] + p_ref[1]) * inv + r_ref[...]


def _make_sc_agg(n_nodes, np_rows, d, nchunk):
    mesh = plsc.VectorSubcoreMesh(core_axis_name="c", subcore_axis_name="s")
    rpt = np_rows // NS

    nbuf = NBUF
    assert nchunk % nbuf == 0 and nchunk // nbuf >= 2

    @functools.partial(
        pl.kernel,
        out_type=jax.ShapeDtypeStruct((NC, np_rows, d), jnp.float32),
        mesh=mesh,
        scratch_types=[
            pltpu.VMEM_SHARED((np_rows, d), jnp.float32),
            pltpu.VMEM((nchunk, CH), jnp.int32),
            pltpu.VMEM((nbuf, CH), jnp.int32),
            pltpu.VMEM((nbuf, CH, d), jnp.float32),
        ] + [pltpu.SemaphoreType.DMA] * (2 * nbuf),
    )
    def sc_agg(m_hbm, src_hbm, dst_hbm, z_hbm, out_hbm,
               acc_sh, src_v, dstr_v, rows_v, *sems):
        rsem = sems[:nbuf]
        isem = sems[nbuf:]
        c = lax.axis_index("c")
        s = lax.axis_index("s")
        wid = s * NC + c
        r0 = s * rpt
        pltpu.sync_copy(z_hbm.at[pl.ds(r0, rpt)], acc_sh.at[pl.ds(r0, rpt)])
        pltpu.sync_copy(src_hbm.at[wid], src_v)
        plsc.subcore_barrier()

        for b in range(nbuf):
            pltpu.async_copy(dst_hbm.at[wid, b], dstr_v.at[b], isem[b])
            pltpu.async_copy(m_hbm.at[src_v.at[b]], rows_v.at[b], rsem[b])

        def step(g, carry):
            j = g * nbuf
            for b in range(nbuf):
                jj = j + b
                pltpu.make_async_copy(
                    dst_hbm.at[wid, jj], dstr_v.at[b], isem[b]).wait()
                pltpu.make_async_copy(
                    m_hbm.at[src_v.at[jj]], rows_v.at[b], rsem[b]).wait()
                pltpu.sync_copy(rows_v.at[b], acc_sh.at[dstr_v.at[b]], add=True)
                pltpu.async_copy(dst_hbm.at[wid, jj + nbuf], dstr_v.at[b],
                                 isem[b])
                pltpu.async_copy(
                    m_hbm.at[src_v.at[jj + nbuf]], rows_v.at[b], rsem[b])
            return carry

        lax.fori_loop(0, nchunk // nbuf - 1, step, 0, unroll=False)
        for b in range(nbuf):
            jj = nchunk - nbuf + b
            pltpu.make_async_copy(
                dst_hbm.at[wid, jj], dstr_v.at[b], isem[b]).wait()
            pltpu.make_async_copy(
                m_hbm.at[src_v.at[jj]], rows_v.at[b], rsem[b]).wait()
            pltpu.sync_copy(rows_v.at[b], acc_sh.at[dstr_v.at[b]], add=True)

        plsc.subcore_barrier()
        pltpu.sync_copy(acc_sh.at[pl.ds(r0, rpt)],
                        out_hbm.at[c, pl.ds(r0, rpt)])

    return sc_agg


def _make_sc_cnt(np_rows, d, nchunk):
    mesh = plsc.VectorSubcoreMesh(core_axis_name="c", subcore_axis_name="s")
    rpt = np_rows // NS

    @functools.partial(
        pl.kernel,
        out_type=jax.ShapeDtypeStruct((NC, np_rows, d), jnp.float32),
        mesh=mesh,
        scratch_types=[
            pltpu.VMEM_SHARED((np_rows, d), jnp.float32),
            pltpu.VMEM((nchunk, CH), jnp.int32),
            pltpu.VMEM((CH, d), jnp.float32),
            pltpu.SemaphoreType.DMA,
        ],
    )
    def sc_cnt(dst_hbm, ones_hbm, z_hbm, out_hbm, cnt_sh, dst_v, ones_v, sem):
        c = lax.axis_index("c")
        s = lax.axis_index("s")
        wid = s * NC + c
        r0 = s * rpt
        pltpu.sync_copy(z_hbm.at[pl.ds(r0, rpt)], cnt_sh.at[pl.ds(r0, rpt)])
        pltpu.sync_copy(dst_hbm.at[wid], dst_v)
        pltpu.sync_copy(ones_hbm, ones_v)
        plsc.subcore_barrier()

        def fire(j, carry):
            pltpu.async_copy(ones_v, cnt_sh.at[dst_v.at[j]], sem, add=True)
            return carry

        lax.fori_loop(0, nchunk, fire, 0, unroll=False)

        def drain(j, carry):
            pltpu.make_async_copy(
                ones_v, cnt_sh.at[dst_v.at[j]], sem).wait()
            return carry

        lax.fori_loop(0, nchunk, drain, 0, unroll=False)
        plsc.subcore_barrier()
        pltpu.sync_copy(cnt_sh.at[pl.ds(r0, rpt)],
                        out_hbm.at[c, pl.ds(r0, rpt)])

    return sc_cnt


def kernel(x, edge_index, Wl0, bl0, Wr0, Wl1, bl1, Wr1, Wl2, bl2, Wr2):
    n, d = x.shape
    e = edge_index.shape[1]

    ew = -(-e // (NW * CH * NBUF)) * CH * NBUF
    ep = ew * NW
    nchunk = ew // CH
    np_rows = -(-(n + LANES) // 1024) * 1024
    blk = 1024
    grid = (-(-n // blk),)

    src = edge_index[0]
    dst = edge_index[1]
    pad = ep - e
    src_p = jnp.concatenate([src, jnp.zeros((pad,), jnp.int32)]).reshape(NW, nchunk, CH)
    core_of_w = (jnp.arange(NW, dtype=jnp.int32) % NC)
    src_p = src_p + (core_of_w * n)[:, None, None]
    dst_p = jnp.concatenate([dst, jnp.full((pad,), n, jnp.int32)]).reshape(NW, nchunk, CH)
    zeros_d = jnp.zeros((np_rows, d), jnp.float32)
    ones_c = jnp.ones((CH, d), jnp.float32)

    sc_agg = _make_sc_agg(n, np_rows, d, nchunk)
    sc_cnt = _make_sc_cnt(np_rows, d, nchunk)

    w_spec = pl.BlockSpec((d, d), lambda i: (0, 0))
    b_spec = pl.BlockSpec((1, d), lambda i: (0, 0))
    h_spec = pl.BlockSpec((blk, d), lambda i: (i, 0))
    p_spec = pl.BlockSpec((NC, blk, d), lambda i: (0, i, 0))
    c_spec = p_spec
    h_sds = jax.ShapeDtypeStruct((n, d), jnp.float32)

    tc_pre = pl.pallas_call(
        _tc_pre_body, grid=grid,
        in_specs=[h_spec, w_spec, w_spec, b_spec],
        out_specs=[h_spec, h_spec],
        out_shape=[h_sds, h_sds],
    )
    tc_mid = pl.pallas_call(
        _tc_mid_body, grid=grid,
        in_specs=[p_spec, c_spec, h_spec, w_spec, w_spec, b_spec],
        out_specs=[h_spec, h_spec],
        out_shape=[h_sds, h_sds],
    )
    tc_fin = pl.pallas_call(
        _tc_fin_body, grid=grid,
        in_specs=[p_spec, c_spec, h_spec],
        out_specs=h_spec,
        out_shape=h_sds,
    )

    cnt = sc_cnt(dst_p, ones_c, zeros_d)

    m, r = tc_pre(x, Wl0, Wr0, bl0.reshape(1, d))
    p = sc_agg(jnp.concatenate([m, m], axis=0), src_p, dst_p, zeros_d)
    m, r = tc_mid(p, cnt, r, Wl1, Wr1, bl1.reshape(1, d))
    p = sc_agg(jnp.concatenate([m, m], axis=0), src_p, dst_p, zeros_d)
    m, r = tc_mid(p, cnt, r, Wl2, Wr2, bl2.reshape(1, d))
    p = sc_agg(jnp.concatenate([m, m], axis=0), src_p, dst_p, zeros_d)
    return tc_fin(p, cnt, r)


# speedup vs baseline: 1.1892x; 1.1892x over previous
import functools

import jax
import jax.numpy as jnp
from jax import lax
from jax.experimental import pallas as pl
from jax.experimental.pallas import tpu as pltpu
from jax.experimental.pallas import tpu_sc as plsc

NC = 2
NS = 16
NW = NC * NS
LANES = 16
CH = 128
SL = 8
NBUF = 2


def _dotT(a, b):
    return lax.dot_general(a, b, (((1,), (1,)), ((), ())),
                           preferred_element_type=jnp.float32)


def _tc_pre_body(x_ref, wl_ref, wr_ref, bl_ref, m_ref, r_ref):
    xb = x_ref[...]
    m_ref[...] = _dotT(xb, wl_ref[...])
    r_ref[...] = _dotT(xb, wr_ref[...]) + bl_ref[...]


def _tc_mid_body(p_ref, cnt_ref, r_ref, wl_ref, wr_ref, bl_ref, m_ref, rn_ref):
    cb = cnt_ref[...]
    cnt = cb[0, :, 0:1] + cb[1, :, 0:1]
    inv = 1.0 / jnp.maximum(cnt, 1.0)
    h = (p_ref[0] + p_ref[1]) * inv + r_ref[...]
    h = jnp.maximum(h, 0.0)
    m_ref[...] = _dotT(h, wl_ref[...])
    rn_ref[...] = _dotT(h, wr_ref[...]) + bl_ref[...]


def _tc_fin_body(p_ref, cnt_ref, r_ref, o_ref):
    cb = cnt_ref[...]
    cnt = cb[0, :, 0:1] + cb[1, :, 0:1]
    inv = 1.0 / jnp.maximum(cnt, 1.0)
    o_ref[...] = (p_ref[0] + p_ref[1]) * inv + r_ref[...]


def _make_sc_agg(np_rows, d, n0, n1):
    mesh = plsc.VectorSubcoreMesh(core_axis_name="c", subcore_axis_name="s")
    rpt = np_rows // NS
    nbuf = NBUF
    assert n0 % (nbuf * SL) == 0 and n1 % (nbuf * SL) == 0 and n1 >= 2 * nbuf

    @functools.partial(
        pl.kernel,
        out_type=jax.ShapeDtypeStruct((NC, np_rows, d), jnp.float32),
        mesh=mesh,
        scratch_types=[
            pltpu.VMEM_SHARED((np_rows, d), jnp.float32),
            pltpu.VMEM((n0 // SL, SL, CH), jnp.int32),
            pltpu.VMEM((nbuf, CH), jnp.int32),
            pltpu.VMEM((nbuf, CH, d), jnp.float32),
        ] + [pltpu.SemaphoreType.DMA] * (2 * nbuf),
    )
    def sc_agg(m_hbm, src_hbm, dst_hbm, z_hbm, out_hbm,
               acc_sh, src_v, dstr_v, rows_v, *sems):
        rsem = sems[:nbuf]
        isem = sems[nbuf:]
        c = lax.axis_index("c")
        s = lax.axis_index("s")
        bslot = jnp.where(c == 0, s * (n0 // SL),
                          NS * (n0 // SL) + s * (n1 // SL))
        cnt = jnp.where(c == 0, n0, n1)
        r0 = s * rpt
        pltpu.sync_copy(z_hbm.at[pl.ds(r0, rpt)], acc_sh.at[pl.ds(r0, rpt)])
        pltpu.sync_copy(src_hbm.at[pl.ds(bslot, n0 // SL)], src_v)
        plsc.subcore_barrier()

        for b in range(nbuf):
            pltpu.async_copy(dst_hbm.at[bslot + b // SL, b % SL],
                             dstr_v.at[b], isem[b])
            pltpu.async_copy(m_hbm.at[src_v.at[b // SL, b % SL]],
                             rows_v.at[b], rsem[b])

        def step(g, carry):
            j = g * nbuf
            for b in range(nbuf):
                jj = j + b
                pltpu.make_async_copy(
                    dst_hbm.at[bslot + jj // SL, jj % SL],
                    dstr_v.at[b], isem[b]).wait()
                pltpu.make_async_copy(
                    m_hbm.at[src_v.at[jj // SL, jj % SL]],
                    rows_v.at[b], rsem[b]).wait()
                pltpu.sync_copy(rows_v.at[b], acc_sh.at[dstr_v.at[b]], add=True)
                jn = jj + nbuf
                pltpu.async_copy(dst_hbm.at[bslot + jn // SL, jn % SL],
                                 dstr_v.at[b], isem[b])
                pltpu.async_copy(m_hbm.at[src_v.at[jn // SL, jn % SL]],
                                 rows_v.at[b], rsem[b])
            return carry

        lax.fori_loop(0, cnt // nbuf - 1, step, 0, unroll=False)
        for b in range(nbuf):
            jj = cnt - nbuf + b
            pltpu.make_async_copy(
                dst_hbm.at[bslot + jj // SL, jj % SL],
                dstr_v.at[b], isem[b]).wait()
            pltpu.make_async_copy(
                m_hbm.at[src_v.at[jj // SL, jj % SL]],
                rows_v.at[b], rsem[b]).wait()
            pltpu.sync_copy(rows_v.at[b], acc_sh.at[dstr_v.at[b]], add=True)

        plsc.subcore_barrier()
        pltpu.sync_copy(acc_sh.at[pl.ds(r0, rpt)],
                        out_hbm.at[c, pl.ds(r0, rpt)])

    return sc_agg


def _make_sc_cnt(np_rows, d, nslot):
    mesh = plsc.VectorSubcoreMesh(core_axis_name="c", subcore_axis_name="s")
    rpt = np_rows // NS

    @functools.partial(
        pl.kernel,
        out_type=jax.ShapeDtypeStruct((NC, np_rows, d), jnp.float32),
        mesh=mesh,
        scratch_types=[
            pltpu.VMEM_SHARED((np_rows, d), jnp.float32),
            pltpu.VMEM((nslot, SL, CH), jnp.int32),
            pltpu.VMEM((CH, d), jnp.float32),
            pltpu.SemaphoreType.DMA,
        ],
    )
    def sc_cnt(dst_hbm, ones_hbm, z_hbm, out_hbm, cnt_sh, dst_v, ones_v, sem):
        c = lax.axis_index("c")
        s = lax.axis_index("s")
        wid = s * NC + c
        r0 = s * rpt
        pltpu.sync_copy(z_hbm.at[pl.ds(r0, rpt)], cnt_sh.at[pl.ds(r0, rpt)])
        pltpu.sync_copy(dst_hbm.at[pl.ds(wid * nslot, nslot)], dst_v)
        pltpu.sync_copy(ones_hbm, ones_v)
        plsc.subcore_barrier()

        def fire(j, carry):
            pltpu.async_copy(ones_v, cnt_sh.at[dst_v.at[j // SL, j % SL]],
                             sem, add=True)
            return carry

        lax.fori_loop(0, nslot * SL, fire, 0, unroll=False)

        def drain(j, carry):
            pltpu.make_async_copy(
                ones_v, cnt_sh.at[dst_v.at[j // SL, j % SL]], sem).wait()
            return carry

        lax.fori_loop(0, nslot * SL, drain, 0, unroll=False)
        plsc.subcore_barrier()
        pltpu.sync_copy(cnt_sh.at[pl.ds(r0, rpt)],
                        out_hbm.at[c, pl.ds(r0, rpt)])

    return sc_cnt


def kernel(x, edge_index, Wl0, bl0, Wr0, Wl1, bl1, Wr1, Wl2, bl2, Wr2):
    n, d = x.shape
    e = edge_index.shape[1]

    unit = 2 * SL * NBUF
    tpp = -(-e // (NS * CH * unit)) * unit
    ratio = 0.785
    n0 = int(round(tpp * ratio / (SL * NBUF))) * SL * NBUF
    n1 = tpp - n0
    nslots = NS * (n0 + n1) // SL
    salloc = nslots + n0 // SL
    assert nslots * SL * CH >= e and nslots % NW == 0 and n1 >= 2 * NBUF
    np_rows = -(-(n + 8) // (8 * NS)) * (8 * NS)
    blk = 1024
    grid = (-(-n // blk),)

    src = edge_index[0]
    dst = edge_index[1]
    pad = salloc * SL * CH - e
    src_p = jnp.concatenate(
        [src, jnp.zeros((pad,), jnp.int32)]).reshape(salloc, SL, CH)
    dst_p = jnp.concatenate(
        [dst, jnp.full((pad,), n, jnp.int32)]).reshape(salloc, SL, CH)
    zeros_d = jnp.zeros((np_rows, d), jnp.float32)
    ones_c = jnp.ones((CH, d), jnp.float32)

    sc_agg = _make_sc_agg(np_rows, d, n0, n1)
    sc_cnt = _make_sc_cnt(np_rows, d, nslots // NW)

    w_spec = pl.BlockSpec((d, d), lambda i: (0, 0))
    b_spec = pl.BlockSpec((1, d), lambda i: (0, 0))
    h_spec = pl.BlockSpec((blk, d), lambda i: (i, 0))
    p_spec = pl.BlockSpec((NC, blk, d), lambda i: (0, i, 0))
    c_spec = p_spec
    h_sds = jax.ShapeDtypeStruct((n, d), jnp.float32)

    tc_pre = pl.pallas_call(
        _tc_pre_body, grid=grid,
        in_specs=[h_spec, w_spec, w_spec, b_spec],
        out_specs=[h_spec, h_spec],
        out_shape=[h_sds, h_sds],
    )
    tc_mid = pl.pallas_call(
        _tc_mid_body, grid=grid,
        in_specs=[p_spec, c_spec, h_spec, w_spec, w_spec, b_spec],
        out_specs=[h_spec, h_spec],
        out_shape=[h_sds, h_sds],
    )
    tc_fin = pl.pallas_call(
        _tc_fin_body, grid=grid,
        in_specs=[p_spec, c_spec, h_spec],
        out_specs=h_spec,
        out_shape=h_sds,
    )

    cnt = sc_cnt(dst_p, ones_c, zeros_d)

    m, r = tc_pre(x, Wl0, Wr0, bl0.reshape(1, d))
    p = sc_agg(m, src_p, dst_p, zeros_d)
    m, r = tc_mid(p, cnt, r, Wl1, Wr1, bl1.reshape(1, d))
    p = sc_agg(m, src_p, dst_p, zeros_d)
    m, r = tc_mid(p, cnt, r, Wl2, Wr2, bl2.reshape(1, d))
    p = sc_agg(m, src_p, dst_p, zeros_d)
    return tc_fin(p, cnt, r)
